# baseline (speedup 1.0000x reference)
import functools

import jax
import jax.numpy as jnp
from jax import lax
from jax.experimental import pallas as pl
from jax.experimental.pallas import tpu as pltpu
from jax.experimental.pallas import tpu_sc as plsc

_B = 4096
_L = 200
_D = 32
_NTOK = _B * _L

_info = plsc.get_sparse_core_info()
_NC = _info.num_cores
_NS = _info.num_subcores
_NW = _NC * _NS
_PER_W = _NTOK // _NW
_CHUNK = 1024
_NSTEP = _PER_W // _CHUNK

_mesh = plsc.VectorSubcoreMesh(core_axis_name="c", subcore_axis_name="s")


@functools.partial(
    pl.kernel,
    mesh=_mesh,
    out_type=jax.ShapeDtypeStruct((_NTOK, _D), jnp.float32),
    scratch_types=[
        pltpu.VMEM((_CHUNK,), jnp.int32),
        pltpu.VMEM((_CHUNK, _D), jnp.float32),
        pltpu.SemaphoreType.DMA,
    ],
    compiler_params=pltpu.CompilerParams(use_tc_tiling_on_sc=False),
)
def _emb_gather(idx_hbm, table_hbm, out_hbm, idx_v, rows_v, sem):
    wid = lax.axis_index("s") * _NC + lax.axis_index("c")
    base = wid * _PER_W

    def step(i, carry):
        off = base + i * _CHUNK
        pltpu.sync_copy(idx_hbm.at[pl.ds(off, _CHUNK)], idx_v)
        pltpu.async_copy(table_hbm.at[idx_v], rows_v, sem).wait()
        pltpu.sync_copy(rows_v, out_hbm.at[pl.ds(off, _CHUNK)])
        return carry

    lax.fori_loop(0, _NSTEP, step, 0)


def kernel(sequence, emb_weight):
    seq_perm = (
        sequence.astype(jnp.int32)
        .reshape(_B // 8, 8, _L // 4, 4)
        .transpose(0, 2, 1, 3)
        .reshape(-1)
    )
    out = _emb_gather(seq_perm, emb_weight)
    return (
        out.reshape(_B // 8, _L // 4, 8, 4 * _D)
        .transpose(0, 2, 1, 3)
        .reshape(_B, _L * _D)
    )


# speedup vs baseline: 1.0826x; 1.0826x over previous
import functools

import jax
import jax.numpy as jnp
from jax import lax
from jax.experimental import pallas as pl
from jax.experimental.pallas import tpu as pltpu
from jax.experimental.pallas import tpu_sc as plsc

_B = 4096
_L = 200
_D = 32
_NTOK = _B * _L

_info = plsc.get_sparse_core_info()
_NC = _info.num_cores
_NS = _info.num_subcores
_NW = _NC * _NS
_PER_W = _NTOK // _NW
_CHUNK = 1280
_NSTEP = _PER_W // _CHUNK

_mesh = plsc.VectorSubcoreMesh(core_axis_name="c", subcore_axis_name="s")


@functools.partial(
    pl.kernel,
    mesh=_mesh,
    out_type=jax.ShapeDtypeStruct((_NTOK, _D), jnp.float32),
    scratch_types=[
        pltpu.VMEM((_PER_W,), jnp.int32),
        pltpu.VMEM((_CHUNK, _D), jnp.float32),
        pltpu.VMEM((_CHUNK, _D), jnp.float32),
        pltpu.SemaphoreType.DMA,
        pltpu.SemaphoreType.DMA,
        pltpu.SemaphoreType.DMA,
        pltpu.SemaphoreType.DMA,
    ],
    compiler_params=pltpu.CompilerParams(use_tc_tiling_on_sc=False),
)
def _emb_gather(idx_hbm, table_hbm, out_hbm, idx_v, rows0, rows1, g0, g1, w0, w1):
    wid = lax.axis_index("s") * _NC + lax.axis_index("c")
    base = wid * _PER_W
    rows = (rows0, rows1)
    gsem = (g0, g1)
    wsem = (w0, w1)

    pltpu.sync_copy(idx_hbm.at[pl.ds(base, _PER_W)], idx_v)

    def gather_start(step, b):
        idx_slice = idx_v.at[pl.ds(step * _CHUNK, _CHUNK)]
        pltpu.async_copy(table_hbm.at[idx_slice], rows[b], gsem[b])

    gather_start(0, 0)
    gather_start(1, 1)

    def body(i0):
        for b in range(2):
            i = i0 + b
            pltpu.make_async_copy(table_hbm.at[idx_v.at[pl.ds(0, _CHUNK)]],
                                  rows[b], gsem[b]).wait()
            pltpu.async_copy(rows[b], out_hbm.at[pl.ds(base + i * _CHUNK, _CHUNK)],
                             wsem[b])

            @pl.when(i < _NSTEP - 2)
            def _():
                pltpu.make_async_copy(
                    rows[b], out_hbm.at[pl.ds(base, _CHUNK)], wsem[b]).wait()
                gather_start(i + 2, b)

    pl.loop(0, _NSTEP, step=2)(body)

    pltpu.make_async_copy(rows[0], out_hbm.at[pl.ds(base, _CHUNK)], w0).wait()
    pltpu.make_async_copy(rows[1], out_hbm.at[pl.ds(base, _CHUNK)], w1).wait()


def kernel(sequence, emb_weight):
    idx = sequence.reshape(-1).astype(jnp.int32)
    out = _emb_gather(idx, emb_weight)
    return out.reshape(sequence.shape[0], -1)
